# Initial kernel scaffold; baseline (speedup 1.0000x reference)
import jax
import jax.numpy as jnp
from jax.experimental import pallas as pl


def kernel(x, batch, W1, b1, W2, b2):
    raise NotImplementedError("write your pallas kernel here")


# speedup vs baseline: 1.6652x; 1.6652x over previous
import functools

import jax
import jax.numpy as jnp
from jax import lax
from jax.experimental import pallas as pl
from jax.experimental.pallas import tpu as pltpu
from jax.experimental.pallas import tpu_sc as plsc

N = 100000
D = 256
G = 512
H = 128
OUT = 128

NC = 2
NS = 16
L = 16
NW = NC * NS
SEG_PER_W = G // NW
C = 160
NT = N // C
DC = D // L


def _lane_bcast(v, i):
    idx = jnp.full((L, 1), i, dtype=jnp.int32)
    dn = lax.GatherDimensionNumbers(
        offset_dims=(), collapsed_slice_dims=(0,), start_index_map=(0,))
    return lax.gather(v, idx, dn, slice_sizes=(1,),
                      mode=lax.GatherScatterMode.PROMISE_IN_BOUNDS)


def _pool_body(x_hbm, batch_hbm, zmean_hbm, zmax_hbm,
               xbuf, bbuf, midbuf, valbuf, sum_acc, max_acc):
    cid = lax.axis_index("c")
    sid = lax.axis_index("s")
    wid = sid * NC + cid
    g0 = wid * SEG_PER_W

    iota = lax.iota(jnp.int32, L)
    zero16 = jnp.zeros((L,), jnp.float32)
    ninf16 = jnp.full((L,), -jnp.inf, jnp.float32)

    def lower_bound(targets):
        def step(_, carry):
            lo, hi = carry
            mid = lax.shift_right_logical(lo + hi, 1)
            midbuf[...] = jnp.minimum(mid, N - 1)
            pltpu.sync_copy(batch_hbm.at[midbuf], valbuf)
            v = valbuf[...]
            pred = (v < targets) & (mid < N)
            lo2 = jnp.where(pred, mid + 1, lo)
            hi2 = jnp.where(pred, hi, mid)
            return lo2, hi2
        lo0 = jnp.zeros((L,), jnp.int32)
        hi0 = jnp.full((L,), N, jnp.int32)
        lo, hi = lax.fori_loop(0, 17, step, (lo0, hi0))
        return lo

    targets = g0 + iota
    starts = lower_bound(targets)
    ends = lower_bound(targets + 1)
    counts = ends - starts

    s_w = jnp.min(starts)
    e_w = jnp.max(ends)

    def init_body(i, _):
        off = i * L
        sum_acc[pl.ds(off, L)] = zero16
        max_acc[pl.ds(off, L)] = ninf16
        return 0
    lax.fori_loop(0, SEG_PER_W * DC, init_body, 0)

    cols = [iota + k * L for k in range(DC)]

    def row_body(j, _):
        grp = (j // L) * L
        vb = bbuf[pl.ds(grp, L)]
        seg = _lane_bcast(vb, j - grp) - g0
        base = seg * D
        for k in range(DC):
            idx = base + cols[k]
            xc = xbuf[pl.ds(j * D + k * L, L)]
            plsc.addupdate_scatter(sum_acc, [idx], xc)
            old = plsc.load_gather(max_acc, [idx])
            plsc.store_scatter(max_acc, [idx], jnp.maximum(old, xc))
        return 0

    def tile_body(t, _):
        pltpu.sync_copy(x_hbm.at[pl.ds(t * (C * D), C * D)], xbuf)
        pltpu.sync_copy(batch_hbm.at[pl.ds(t * C, C)], bbuf)
        j_lo = jnp.maximum(s_w - t * C, 0)
        j_hi = jnp.minimum(e_w - t * C, C)
        lax.fori_loop(j_lo, j_hi, row_body, 0)
        return 0

    t0 = s_w // C
    t1 = (e_w + (C - 1)) // C
    lax.fori_loop(t0, t1, tile_body, 0)

    countf = counts.astype(jnp.float32)

    def fin_body(g, _):
        cnt = _lane_bcast(countf, g)
        ne = cnt > 0.0
        safe = jnp.maximum(cnt, 1.0)
        for k in range(DC):
            off = g * D + k * L
            s = sum_acc[pl.ds(off, L)]
            m = max_acc[pl.ds(off, L)]
            sum_acc[pl.ds(off, L)] = jnp.where(ne, s / safe, 0.0)
            max_acc[pl.ds(off, L)] = jnp.where(ne, m, 0.0)
        return 0
    lax.fori_loop(0, SEG_PER_W, fin_body, 0)

    pltpu.sync_copy(sum_acc, zmean_hbm.at[pl.ds(g0 * D, SEG_PER_W * D)])
    pltpu.sync_copy(max_acc, zmax_hbm.at[pl.ds(g0 * D, SEG_PER_W * D)])


@jax.jit
def _pool(xf, batch):
    mesh = plsc.VectorSubcoreMesh(core_axis_name="c", subcore_axis_name="s",
                                  num_cores=NC, num_subcores=NS)
    return pl.kernel(
        _pool_body,
        compiler_params=pltpu.CompilerParams(needs_layout_passes=False),
        out_type=(
            jax.ShapeDtypeStruct((G * D,), jnp.float32),
            jax.ShapeDtypeStruct((G * D,), jnp.float32),
        ),
        mesh=mesh,
        scratch_types=[
            pltpu.VMEM((C * D,), jnp.float32),
            pltpu.VMEM((C,), jnp.int32),
            pltpu.VMEM((L,), jnp.int32),
            pltpu.VMEM((L,), jnp.int32),
            pltpu.VMEM((SEG_PER_W * D,), jnp.float32),
            pltpu.VMEM((SEG_PER_W * D,), jnp.float32),
        ],
    )(xf, batch)


def _dense_body(zm_ref, zx_ref, w1_ref, b1_ref, w2_ref, b2_ref, o_ref):
    dn = (((1,), (1,)), ((), ()))
    hm = lax.dot_general(zm_ref[...], w1_ref[...], dn,
                         preferred_element_type=jnp.float32) + b1_ref[...]
    hx = lax.dot_general(zx_ref[...], w1_ref[...], dn,
                         preferred_element_type=jnp.float32) + b1_ref[...]
    h = jnp.concatenate([hm, hx], axis=1)
    o_ref[...] = lax.dot_general(h, w2_ref[...], dn,
                                 preferred_element_type=jnp.float32) + b2_ref[...]


@jax.jit
def _dense(zm, zx, W1, b1, W2, b2):
    return pl.pallas_call(
        _dense_body,
        out_shape=jax.ShapeDtypeStruct((G, OUT), jnp.float32),
    )(zm, zx, W1, b1, W2, b2)


def kernel(x, batch, W1, b1, W2, b2):
    zm_flat, zx_flat = _pool(x.reshape(-1), batch)
    zm = zm_flat.reshape(G, D)
    zx = zx_flat.reshape(G, D)
    return _dense(zm, zx, W1, b1.reshape(1, H), W2, b2.reshape(1, OUT))


# speedup vs baseline: 3.6729x; 2.2057x over previous
import functools

import jax
import jax.numpy as jnp
from jax import lax
from jax.experimental import pallas as pl
from jax.experimental.pallas import tpu as pltpu
from jax.experimental.pallas import tpu_sc as plsc

N = 100000
D = 256
G = 512
H = 128
OUT = 128

NC = 2
NS = 16
L = 16
NW = NC * NS
SEG_PER_W = G // NW
C = 160
NT = N // C
DC = D // L


def _lane_bcast(v, i):
    idx = jnp.full((L, 1), i, dtype=jnp.int32)
    dn = lax.GatherDimensionNumbers(
        offset_dims=(), collapsed_slice_dims=(0,), start_index_map=(0,))
    return lax.gather(v, idx, dn, slice_sizes=(1,),
                      mode=lax.GatherScatterMode.PROMISE_IN_BOUNDS)


def _pool_body(x_hbm, batch_hbm, zmean_hbm, zmax_hbm,
               xbuf, bbuf, midbuf, valbuf, sum_acc, max_acc,
               cur_sum, cur_max):
    cid = lax.axis_index("c")
    sid = lax.axis_index("s")
    wid = sid * NC + cid
    g0 = wid * SEG_PER_W
    g0v = jnp.full((L,), 0, jnp.int32) + g0

    iota = lax.iota(jnp.int32, L)
    zero16 = jnp.zeros((L,), jnp.float32)
    ninf16 = jnp.full((L,), -jnp.inf, jnp.float32)

    def lower_bound(targets):
        def step(_, carry):
            lo, hi = carry
            mid = lax.shift_right_logical(lo + hi, 1)
            midbuf[...] = jnp.minimum(mid, N - 1)
            pltpu.sync_copy(batch_hbm.at[midbuf], valbuf)
            v = valbuf[...]
            pred = (v < targets) & (mid < N)
            lo2 = jnp.where(pred, mid + 1, lo)
            hi2 = jnp.where(pred, hi, mid)
            return lo2, hi2
        lo0 = jnp.zeros((L,), jnp.int32)
        hi0 = jnp.full((L,), N, jnp.int32)
        lo, hi = lax.fori_loop(0, 17, step, (lo0, hi0))
        return lo

    targets = g0 + iota
    starts = lower_bound(targets)
    ends = lower_bound(targets + 1)
    counts = ends - starts

    s_w = jnp.min(starts)
    e_w = jnp.max(ends)

    def init_body(i, _):
        off = i * L
        sum_acc[pl.ds(off, L)] = zero16
        max_acc[pl.ds(off, L)] = ninf16
        return 0
    lax.fori_loop(0, SEG_PER_W * DC, init_body, 0)
    for k in range(DC):
        cur_sum[pl.ds(k * L, L)] = zero16
        cur_max[pl.ds(k * L, L)] = ninf16

    cols = [iota + k * L for k in range(DC)]

    def flush_cur(cur_gv):
        mk = cur_gv >= 0
        base = (cur_gv - g0v) * D
        for k in range(DC):
            idx = base + cols[k]
            plsc.addupdate_scatter(sum_acc, [idx], cur_sum[pl.ds(k * L, L)],
                                   mask=mk)
            old = plsc.load_gather(max_acc, [idx], mask=mk)
            plsc.store_scatter(
                max_acc, [idx],
                jnp.maximum(old, cur_max[pl.ds(k * L, L)]), mask=mk)

    def group_body(gi, cur_gv):
        vb = bbuf[pl.ds(gi * L, L)]
        goff = gi * (L * D)

        def fast(cur_gv):
            for k in range(DC):
                xs = [xbuf[pl.ds(goff + r * D + k * L, L)] for r in range(L)]
                ss = xs
                while len(ss) > 1:
                    ss = [ss[2 * i] + ss[2 * i + 1] for i in range(len(ss) // 2)]
                ms = xs
                while len(ms) > 1:
                    ms = [jnp.maximum(ms[2 * i], ms[2 * i + 1])
                          for i in range(len(ms) // 2)]
                off = k * L
                cur_sum[pl.ds(off, L)] = cur_sum[pl.ds(off, L)] + ss[0]
                cur_max[pl.ds(off, L)] = jnp.maximum(cur_max[pl.ds(off, L)],
                                                     ms[0])
            return cur_gv

        def slow(cur_gv):
            flush_cur(cur_gv)
            for k in range(DC):
                cur_sum[pl.ds(k * L, L)] = zero16
                cur_max[pl.ds(k * L, L)] = ninf16

            def row_body(r, _):
                seg = _lane_bcast(vb, r) - g0v
                base = seg * D
                for k in range(DC):
                    idx = base + cols[k]
                    xc = xbuf[pl.ds(goff + r * D + k * L, L)]
                    plsc.addupdate_scatter(sum_acc, [idx], xc)
                    old = plsc.load_gather(max_acc, [idx])
                    plsc.store_scatter(max_acc, [idx], jnp.maximum(old, xc))
                return 0
            lax.fori_loop(r_lo, r_hi, row_body, 0)
            return _lane_bcast(vb, r_hi - 1)

        r_lo = jnp.maximum(j_lo - gi * L, 0)
        r_hi = jnp.minimum(j_hi - gi * L, L)
        is_full = (r_lo == 0) & (r_hi == L)
        take_fast = is_full & jnp.all(vb == cur_gv)
        return lax.cond(take_fast, fast, slow, cur_gv)

    j_lo = jnp.int32(0)
    j_hi = jnp.int32(0)

    def tile_body2(t, cur_gv):
        nonlocal j_lo, j_hi
        pltpu.sync_copy(x_hbm.at[pl.ds(t * (C * D), C * D)], xbuf)
        pltpu.sync_copy(batch_hbm.at[pl.ds(t * C, C)], bbuf)
        j_lo = jnp.maximum(s_w - t * C, 0)
        j_hi = jnp.minimum(e_w - t * C, C)
        gi_lo = j_lo // L
        gi_hi = (j_hi + (L - 1)) // L
        return lax.fori_loop(gi_lo, gi_hi, group_body, cur_gv)

    t0 = s_w // C
    t1 = (e_w + (C - 1)) // C
    t1 = jnp.where(s_w == e_w, t0, t1)
    cur_gv = lax.fori_loop(t0, t1, tile_body2,
                           jnp.full((L,), -1, jnp.int32))
    flush_cur(cur_gv)

    countf = counts.astype(jnp.float32)

    def fin_body(g, _):
        cnt = _lane_bcast(countf, g)
        ne = cnt > 0.0
        safe = jnp.maximum(cnt, 1.0)
        for k in range(DC):
            off = g * D + k * L
            s = sum_acc[pl.ds(off, L)]
            m = max_acc[pl.ds(off, L)]
            sum_acc[pl.ds(off, L)] = jnp.where(ne, s / safe, 0.0)
            max_acc[pl.ds(off, L)] = jnp.where(ne, m, 0.0)
        return 0
    lax.fori_loop(0, SEG_PER_W, fin_body, 0)

    pltpu.sync_copy(sum_acc, zmean_hbm.at[pl.ds(g0 * D, SEG_PER_W * D)])
    pltpu.sync_copy(max_acc, zmax_hbm.at[pl.ds(g0 * D, SEG_PER_W * D)])


@jax.jit
def _pool(xf, batch):
    mesh = plsc.VectorSubcoreMesh(core_axis_name="c", subcore_axis_name="s",
                                  num_cores=NC, num_subcores=NS)
    return pl.kernel(
        _pool_body,
        compiler_params=pltpu.CompilerParams(needs_layout_passes=False),
        out_type=(
            jax.ShapeDtypeStruct((G * D,), jnp.float32),
            jax.ShapeDtypeStruct((G * D,), jnp.float32),
        ),
        mesh=mesh,
        scratch_types=[
            pltpu.VMEM((C * D,), jnp.float32),
            pltpu.VMEM((C,), jnp.int32),
            pltpu.VMEM((L,), jnp.int32),
            pltpu.VMEM((L,), jnp.int32),
            pltpu.VMEM((SEG_PER_W * D,), jnp.float32),
            pltpu.VMEM((SEG_PER_W * D,), jnp.float32),
            pltpu.VMEM((D,), jnp.float32),
            pltpu.VMEM((D,), jnp.float32),
        ],
    )(xf, batch)


def _dense_body(zm_ref, zx_ref, w1_ref, b1_ref, w2_ref, b2_ref, o_ref):
    dn = (((1,), (1,)), ((), ()))
    hm = lax.dot_general(zm_ref[...], w1_ref[...], dn,
                         preferred_element_type=jnp.float32) + b1_ref[...]
    hx = lax.dot_general(zx_ref[...], w1_ref[...], dn,
                         preferred_element_type=jnp.float32) + b1_ref[...]
    h = jnp.concatenate([hm, hx], axis=1)
    o_ref[...] = lax.dot_general(h, w2_ref[...], dn,
                                 preferred_element_type=jnp.float32) + b2_ref[...]


@jax.jit
def _dense(zm, zx, W1, b1, W2, b2):
    return pl.pallas_call(
        _dense_body,
        out_shape=jax.ShapeDtypeStruct((G, OUT), jnp.float32),
    )(zm, zx, W1, b1, W2, b2)


def kernel(x, batch, W1, b1, W2, b2):
    zm_flat, zx_flat = _pool(x.reshape(-1), batch)
    zm = zm_flat.reshape(G, D)
    zx = zx_flat.reshape(G, D)
    return _dense(zm, zx, W1, b1.reshape(1, H), W2, b2.reshape(1, OUT))


# speedup vs baseline: 4.4946x; 1.2237x over previous
import functools

import jax
import jax.numpy as jnp
from jax import lax
from jax.experimental import pallas as pl
from jax.experimental.pallas import tpu as pltpu
from jax.experimental.pallas import tpu_sc as plsc

N = 100000
D = 256
G = 512
H = 128
OUT = 128

NC = 2
NS = 16
L = 16
NW = NC * NS
SEG_PER_W = G // NW
C = 160
NT = N // C
DC = D // L


def _lane_bcast(v, i):
    idx = jnp.full((L, 1), i, dtype=jnp.int32)
    dn = lax.GatherDimensionNumbers(
        offset_dims=(), collapsed_slice_dims=(0,), start_index_map=(0,))
    return lax.gather(v, idx, dn, slice_sizes=(1,),
                      mode=lax.GatherScatterMode.PROMISE_IN_BOUNDS)


def _pool_body(x_hbm, batch_hbm, zmean_hbm, zmax_hbm,
               xbuf, bbuf, midbuf, valbuf, sum_acc, max_acc,
               cur_sum, cur_max, sem0, sem1):
    cid = lax.axis_index("c")
    sid = lax.axis_index("s")
    wid = sid * NC + cid
    g0 = wid * SEG_PER_W
    g0v = jnp.full((L,), 0, jnp.int32) + g0

    iota = lax.iota(jnp.int32, L)
    zero16 = jnp.zeros((L,), jnp.float32)
    ninf16 = jnp.full((L,), -jnp.inf, jnp.float32)

    def lower_bound(targets):
        def step(_, carry):
            lo, hi = carry
            mid = lax.shift_right_logical(lo + hi, 1)
            midbuf[...] = jnp.minimum(mid, N - 1)
            pltpu.sync_copy(batch_hbm.at[midbuf], valbuf)
            v = valbuf[...]
            pred = (v < targets) & (mid < N)
            lo2 = jnp.where(pred, mid + 1, lo)
            hi2 = jnp.where(pred, hi, mid)
            return lo2, hi2
        lo0 = jnp.zeros((L,), jnp.int32)
        hi0 = jnp.full((L,), N, jnp.int32)
        lo, hi = lax.fori_loop(0, 17, step, (lo0, hi0))
        return lo

    targets = g0 + iota
    starts = lower_bound(targets)
    ends = lower_bound(targets + 1)
    counts = ends - starts

    s_w = jnp.min(starts)
    e_w = jnp.max(ends)

    def init_body(i, _):
        off = i * L
        sum_acc[pl.ds(off, L)] = zero16
        max_acc[pl.ds(off, L)] = ninf16
        return 0
    lax.fori_loop(0, SEG_PER_W * DC, init_body, 0)
    for k in range(DC):
        cur_sum[pl.ds(k * L, L)] = zero16
        cur_max[pl.ds(k * L, L)] = ninf16

    cols = [iota + k * L for k in range(DC)]

    def flush_cur(cur_gv):
        mk = cur_gv >= 0
        base = (cur_gv - g0v) * D
        for k in range(DC):
            idx = base + cols[k]
            plsc.addupdate_scatter(sum_acc, [idx], cur_sum[pl.ds(k * L, L)],
                                   mask=mk)
            old = plsc.load_gather(max_acc, [idx], mask=mk)
            plsc.store_scatter(
                max_acc, [idx],
                jnp.maximum(old, cur_max[pl.ds(k * L, L)]), mask=mk)

    def group_body(gi, cur_gv):
        vb = bbuf[pl.ds(pbase_b + gi * L, L)]
        goff = pbase_x + gi * (L * D)

        def fast(cur_gv):
            for k in range(DC):
                xs = [xbuf[pl.ds(goff + r * D + k * L, L)] for r in range(L)]
                ss = xs
                while len(ss) > 1:
                    ss = [ss[2 * i] + ss[2 * i + 1] for i in range(len(ss) // 2)]
                ms = xs
                while len(ms) > 1:
                    ms = [jnp.maximum(ms[2 * i], ms[2 * i + 1])
                          for i in range(len(ms) // 2)]
                off = k * L
                cur_sum[pl.ds(off, L)] = cur_sum[pl.ds(off, L)] + ss[0]
                cur_max[pl.ds(off, L)] = jnp.maximum(cur_max[pl.ds(off, L)],
                                                     ms[0])
            return cur_gv

        def slow(cur_gv):
            flush_cur(cur_gv)
            for k in range(DC):
                cur_sum[pl.ds(k * L, L)] = zero16
                cur_max[pl.ds(k * L, L)] = ninf16

            def row_body(r, _):
                seg = _lane_bcast(vb, r) - g0v
                base = seg * D
                for k in range(DC):
                    idx = base + cols[k]
                    xc = xbuf[pl.ds(goff + r * D + k * L, L)]
                    plsc.addupdate_scatter(sum_acc, [idx], xc)
                    old = plsc.load_gather(max_acc, [idx])
                    plsc.store_scatter(max_acc, [idx], jnp.maximum(old, xc))
                return 0
            lax.fori_loop(r_lo, r_hi, row_body, 0)
            return _lane_bcast(vb, r_hi - 1)

        r_lo = jnp.maximum(j_lo - gi * L, 0)
        r_hi = jnp.minimum(j_hi - gi * L, L)
        is_full = (r_lo == 0) & (r_hi == L)
        take_fast = is_full & jnp.all(vb == cur_gv)
        return lax.cond(take_fast, fast, slow, cur_gv)

    j_lo = jnp.int32(0)
    j_hi = jnp.int32(0)
    pbase_x = jnp.int32(0)
    pbase_b = jnp.int32(0)

    CD = C * D
    t0 = s_w // C
    t1 = (e_w + (C - 1)) // C
    t1 = jnp.where(s_w == e_w, t0, t1)

    def start_tile(t, slot, sem):
        pltpu.async_copy(x_hbm.at[pl.ds(t * CD, CD)],
                         xbuf.at[pl.ds(slot * CD, CD)], sem)
        pltpu.async_copy(batch_hbm.at[pl.ds(t * C, C)],
                         bbuf.at[pl.ds(slot * C, C)], sem)

    def wait_tile(t, slot, sem):
        pltpu.make_async_copy(x_hbm.at[pl.ds(t * CD, CD)],
                              xbuf.at[pl.ds(slot * CD, CD)], sem).wait()
        pltpu.make_async_copy(batch_hbm.at[pl.ds(t * C, C)],
                              bbuf.at[pl.ds(slot * C, C)], sem).wait()

    @pl.when(t0 < t1)
    def _():
        start_tile(t0, 0, sem0)

    def tile_body2(t, cur_gv):
        nonlocal j_lo, j_hi, pbase_x, pbase_b
        parity = (t - t0) & 1

        @pl.when(parity == 0)
        def _():
            wait_tile(t, 0, sem0)

            @pl.when(t + 1 < t1)
            def _():
                start_tile(t + 1, 1, sem1)

        @pl.when(parity == 1)
        def _():
            wait_tile(t, 1, sem1)

            @pl.when(t + 1 < t1)
            def _():
                start_tile(t + 1, 0, sem0)

        j_lo = jnp.maximum(s_w - t * C, 0)
        j_hi = jnp.minimum(e_w - t * C, C)
        pbase_x = parity * CD
        pbase_b = parity * C
        gi_lo = j_lo // L
        gi_hi = (j_hi + (L - 1)) // L
        return lax.fori_loop(gi_lo, gi_hi, group_body, cur_gv)

    cur_gv = lax.fori_loop(t0, t1, tile_body2,
                           jnp.full((L,), -1, jnp.int32))
    flush_cur(cur_gv)

    countf = counts.astype(jnp.float32)

    def fin_body(g, _):
        cnt = _lane_bcast(countf, g)
        ne = cnt > 0.0
        safe = jnp.maximum(cnt, 1.0)
        for k in range(DC):
            off = g * D + k * L
            s = sum_acc[pl.ds(off, L)]
            m = max_acc[pl.ds(off, L)]
            sum_acc[pl.ds(off, L)] = jnp.where(ne, s / safe, 0.0)
            max_acc[pl.ds(off, L)] = jnp.where(ne, m, 0.0)
        return 0
    lax.fori_loop(0, SEG_PER_W, fin_body, 0)

    pltpu.sync_copy(sum_acc, zmean_hbm.at[pl.ds(g0 * D, SEG_PER_W * D)])
    pltpu.sync_copy(max_acc, zmax_hbm.at[pl.ds(g0 * D, SEG_PER_W * D)])


@jax.jit
def _pool(xf, batch):
    mesh = plsc.VectorSubcoreMesh(core_axis_name="c", subcore_axis_name="s",
                                  num_cores=NC, num_subcores=NS)
    return pl.kernel(
        _pool_body,
        compiler_params=pltpu.CompilerParams(needs_layout_passes=False),
        out_type=(
            jax.ShapeDtypeStruct((G * D,), jnp.float32),
            jax.ShapeDtypeStruct((G * D,), jnp.float32),
        ),
        mesh=mesh,
        scratch_types=[
            pltpu.VMEM((2 * C * D,), jnp.float32),
            pltpu.VMEM((2 * C,), jnp.int32),
            pltpu.VMEM((L,), jnp.int32),
            pltpu.VMEM((L,), jnp.int32),
            pltpu.VMEM((SEG_PER_W * D,), jnp.float32),
            pltpu.VMEM((SEG_PER_W * D,), jnp.float32),
            pltpu.VMEM((D,), jnp.float32),
            pltpu.VMEM((D,), jnp.float32),
            pltpu.SemaphoreType.DMA,
            pltpu.SemaphoreType.DMA,
        ],
    )(xf, batch)


def _dense_body(zm_ref, zx_ref, w1_ref, b1_ref, w2_ref, b2_ref, o_ref):
    dn = (((1,), (1,)), ((), ()))
    hm = lax.dot_general(zm_ref[...], w1_ref[...], dn,
                         preferred_element_type=jnp.float32) + b1_ref[...]
    hx = lax.dot_general(zx_ref[...], w1_ref[...], dn,
                         preferred_element_type=jnp.float32) + b1_ref[...]
    h = jnp.concatenate([hm, hx], axis=1)
    o_ref[...] = lax.dot_general(h, w2_ref[...], dn,
                                 preferred_element_type=jnp.float32) + b2_ref[...]


@jax.jit
def _dense(zm, zx, W1, b1, W2, b2):
    return pl.pallas_call(
        _dense_body,
        out_shape=jax.ShapeDtypeStruct((G, OUT), jnp.float32),
    )(zm, zx, W1, b1, W2, b2)


def kernel(x, batch, W1, b1, W2, b2):
    zm_flat, zx_flat = _pool(x.reshape(-1), batch)
    zm = zm_flat.reshape(G, D)
    zx = zx_flat.reshape(G, D)
    return _dense(zm, zx, W1, b1.reshape(1, H), W2, b2.reshape(1, OUT))


# speedup vs baseline: 6.7748x; 1.5073x over previous
import functools

import jax
import jax.numpy as jnp
from jax import lax
from jax.experimental import pallas as pl
from jax.experimental.pallas import tpu as pltpu
from jax.experimental.pallas import tpu_sc as plsc

N = 100000
D = 256
G = 512
H = 128
OUT = 128

NC = 2
NS = 16
L = 16
NW = NC * NS
SEG_PER_W = G // NW
C = 160
TPB = C // 8
NT = N // C
DC = D // L


def _lane_bcast(v, i):
    idx = jnp.full((L, 1), i, dtype=jnp.int32)
    dn = lax.GatherDimensionNumbers(
        offset_dims=(), collapsed_slice_dims=(0,), start_index_map=(0,))
    return lax.gather(v, idx, dn, slice_sizes=(1,),
                      mode=lax.GatherScatterMode.PROMISE_IN_BOUNDS)


def _pool_body(x_hbm, batch_hbm, zmean_hbm, zmax_hbm,
               xbuf, bbuf, midbuf, valbuf, sum_acc, max_acc,
               cur_sum, cur_max, sem0, sem1):
    cid = lax.axis_index("c")
    sid = lax.axis_index("s")
    wid = sid * NC + cid
    g0 = wid * SEG_PER_W
    g0v = jnp.full((L,), 0, jnp.int32) + g0

    iota = lax.iota(jnp.int32, L)
    zero16 = jnp.zeros((L,), jnp.float32)
    ninf16 = jnp.full((L,), -jnp.inf, jnp.float32)

    def lower_bound(targets):
        def step(_, carry):
            lo, hi = carry
            mid = lax.shift_right_logical(lo + hi, 1)
            midbuf[...] = jnp.minimum(mid, N - 1)
            pltpu.sync_copy(batch_hbm.at[midbuf], valbuf)
            v = valbuf[...]
            pred = (v < targets) & (mid < N)
            lo2 = jnp.where(pred, mid + 1, lo)
            hi2 = jnp.where(pred, hi, mid)
            return lo2, hi2
        lo0 = jnp.zeros((L,), jnp.int32)
        hi0 = jnp.full((L,), N, jnp.int32)
        lo, hi = lax.fori_loop(0, 17, step, (lo0, hi0))
        return lo

    targets = g0 + iota
    starts = lower_bound(targets)
    ends = lower_bound(targets + 1)
    counts = ends - starts

    s_w = jnp.min(starts)
    e_w = jnp.max(ends)

    def init_body(i, _):
        off = i * L
        sum_acc[pl.ds(off, L)] = zero16
        max_acc[pl.ds(off, L)] = ninf16
        return 0
    lax.fori_loop(0, SEG_PER_W * DC, init_body, 0)
    for k in range(DC):
        cur_sum[pl.ds(k * L, L)] = zero16
        cur_max[pl.ds(k * L, L)] = ninf16

    cols = [iota + k * L for k in range(DC)]

    def flush_cur(cur_gv):
        mk = cur_gv >= 0
        base = (cur_gv - g0v) * D
        for k in range(DC):
            idx = base + cols[k]
            plsc.addupdate_scatter(sum_acc, [idx], cur_sum[pl.ds(k * L, L)],
                                   mask=mk)
            old = plsc.load_gather(max_acc, [idx], mask=mk)
            plsc.store_scatter(
                max_acc, [idx],
                jnp.maximum(old, cur_max[pl.ds(k * L, L)]), mask=mk)

    def group_body(gi, cur_gv):
        vb = bbuf[pl.ds(pbase_b + gi * L, L)]
        gb0 = pbase_rb + gi * 2

        def fast(cur_gv):
            for k in range(DC):
                xs = [xbuf[gb0 + (r >> 3), r & 7, pl.ds(k * L, L)]
                      for r in range(L)]
                ss = xs
                while len(ss) > 1:
                    ss = [ss[2 * i] + ss[2 * i + 1] for i in range(len(ss) // 2)]
                ms = xs
                while len(ms) > 1:
                    ms = [jnp.maximum(ms[2 * i], ms[2 * i + 1])
                          for i in range(len(ms) // 2)]
                off = k * L
                cur_sum[pl.ds(off, L)] = cur_sum[pl.ds(off, L)] + ss[0]
                cur_max[pl.ds(off, L)] = jnp.maximum(cur_max[pl.ds(off, L)],
                                                     ms[0])
            return cur_gv

        def slow(cur_gv):
            flush_cur(cur_gv)
            for k in range(DC):
                cur_sum[pl.ds(k * L, L)] = zero16
                cur_max[pl.ds(k * L, L)] = ninf16

            def row_body(r, _):
                seg = _lane_bcast(vb, r) - g0v
                base = seg * D
                rbi = gb0 + lax.shift_right_logical(r, 3)
                sub = jnp.bitwise_and(r, 7)
                for k in range(DC):
                    idx = base + cols[k]
                    xc = xbuf[rbi, sub, pl.ds(k * L, L)]
                    plsc.addupdate_scatter(sum_acc, [idx], xc)
                    old = plsc.load_gather(max_acc, [idx])
                    plsc.store_scatter(max_acc, [idx], jnp.maximum(old, xc))
                return 0
            lax.fori_loop(r_lo, r_hi, row_body, 0)
            return _lane_bcast(vb, r_hi - 1)

        r_lo = jnp.maximum(j_lo - gi * L, 0)
        r_hi = jnp.minimum(j_hi - gi * L, L)
        is_full = (r_lo == 0) & (r_hi == L)
        take_fast = is_full & jnp.all(vb == cur_gv)
        return lax.cond(take_fast, fast, slow, cur_gv)

    j_lo = jnp.int32(0)
    j_hi = jnp.int32(0)
    pbase_rb = jnp.int32(0)
    pbase_b = jnp.int32(0)

    CD = C * D
    t0 = s_w // C
    t1 = (e_w + (C - 1)) // C
    t1 = jnp.where(s_w == e_w, t0, t1)

    def start_tile(t, slot, sem):
        pltpu.async_copy(x_hbm.at[pl.ds(t * TPB, TPB)],
                         xbuf.at[pl.ds(slot * TPB, TPB)], sem)
        pltpu.async_copy(batch_hbm.at[pl.ds(t * C, C)],
                         bbuf.at[pl.ds(slot * C, C)], sem)

    def wait_tile(t, slot, sem):
        pltpu.make_async_copy(x_hbm.at[pl.ds(t * TPB, TPB)],
                              xbuf.at[pl.ds(slot * TPB, TPB)], sem).wait()
        pltpu.make_async_copy(batch_hbm.at[pl.ds(t * C, C)],
                              bbuf.at[pl.ds(slot * C, C)], sem).wait()

    @pl.when(t0 < t1)
    def _():
        start_tile(t0, 0, sem0)

    def tile_body2(t, cur_gv):
        nonlocal j_lo, j_hi, pbase_rb, pbase_b
        parity = (t - t0) & 1

        @pl.when(parity == 0)
        def _():
            wait_tile(t, 0, sem0)

            @pl.when(t + 1 < t1)
            def _():
                start_tile(t + 1, 1, sem1)

        @pl.when(parity == 1)
        def _():
            wait_tile(t, 1, sem1)

            @pl.when(t + 1 < t1)
            def _():
                start_tile(t + 1, 0, sem0)

        j_lo = jnp.maximum(s_w - t * C, 0)
        j_hi = jnp.minimum(e_w - t * C, C)
        pbase_rb = parity * TPB
        pbase_b = parity * C
        gi_lo = j_lo // L
        gi_hi = (j_hi + (L - 1)) // L
        return lax.fori_loop(gi_lo, gi_hi, group_body, cur_gv)

    cur_gv = lax.fori_loop(t0, t1, tile_body2,
                           jnp.full((L,), -1, jnp.int32))
    flush_cur(cur_gv)

    countf = counts.astype(jnp.float32)

    def fin_body(g, _):
        cnt = _lane_bcast(countf, g)
        ne = cnt > 0.0
        safe = jnp.maximum(cnt, 1.0)
        for k in range(DC):
            off = g * D + k * L
            s = sum_acc[pl.ds(off, L)]
            m = max_acc[pl.ds(off, L)]
            sum_acc[pl.ds(off, L)] = jnp.where(ne, s / safe, 0.0)
            max_acc[pl.ds(off, L)] = jnp.where(ne, m, 0.0)
        return 0
    lax.fori_loop(0, SEG_PER_W, fin_body, 0)

    pltpu.sync_copy(sum_acc, zmean_hbm.at[pl.ds(g0 * D, SEG_PER_W * D)])
    pltpu.sync_copy(max_acc, zmax_hbm.at[pl.ds(g0 * D, SEG_PER_W * D)])


@jax.jit
def _pool(xf, batch):
    mesh = plsc.VectorSubcoreMesh(core_axis_name="c", subcore_axis_name="s",
                                  num_cores=NC, num_subcores=NS)
    return pl.kernel(
        _pool_body,
        compiler_params=pltpu.CompilerParams(needs_layout_passes=False),
        out_type=(
            jax.ShapeDtypeStruct((G * D,), jnp.float32),
            jax.ShapeDtypeStruct((G * D,), jnp.float32),
        ),
        mesh=mesh,
        scratch_types=[
            pltpu.VMEM((2 * TPB, 8, D), jnp.float32),
            pltpu.VMEM((2 * C,), jnp.int32),
            pltpu.VMEM((L,), jnp.int32),
            pltpu.VMEM((L,), jnp.int32),
            pltpu.VMEM((SEG_PER_W * D,), jnp.float32),
            pltpu.VMEM((SEG_PER_W * D,), jnp.float32),
            pltpu.VMEM((D,), jnp.float32),
            pltpu.VMEM((D,), jnp.float32),
            pltpu.SemaphoreType.DMA,
            pltpu.SemaphoreType.DMA,
        ],
    )(xf, batch)


def _dense_body(zm_ref, zx_ref, w1_ref, b1_ref, w2_ref, b2_ref, o_ref):
    dn = (((1,), (1,)), ((), ()))
    hm = lax.dot_general(zm_ref[...], w1_ref[...], dn,
                         preferred_element_type=jnp.float32) + b1_ref[...]
    hx = lax.dot_general(zx_ref[...], w1_ref[...], dn,
                         preferred_element_type=jnp.float32) + b1_ref[...]
    h = jnp.concatenate([hm, hx], axis=1)
    o_ref[...] = lax.dot_general(h, w2_ref[...], dn,
                                 preferred_element_type=jnp.float32) + b2_ref[...]


@jax.jit
def _dense(zm, zx, W1, b1, W2, b2):
    return pl.pallas_call(
        _dense_body,
        out_shape=jax.ShapeDtypeStruct((G, OUT), jnp.float32),
    )(zm, zx, W1, b1, W2, b2)


def kernel(x, batch, W1, b1, W2, b2):
    zm_flat, zx_flat = _pool(x.reshape(N // 8, 8, D), batch)
    zm = zm_flat.reshape(G, D)
    zx = zx_flat.reshape(G, D)
    return _dense(zm, zx, W1, b1.reshape(1, H), W2, b2.reshape(1, OUT))


# speedup vs baseline: 7.0151x; 1.0355x over previous
import functools

import jax
import jax.numpy as jnp
from jax import lax
from jax.experimental import pallas as pl
from jax.experimental.pallas import tpu as pltpu
from jax.experimental.pallas import tpu_sc as plsc

N = 100000
D = 256
G = 512
H = 128
OUT = 128

NC = 2
NS = 16
L = 16
NW = NC * NS
SEG_PER_W = G // NW
C = 160
TPB = C // 8
NT = N // C
DC = D // L


def _lane_bcast(v, i):
    idx = jnp.full((L, 1), i, dtype=jnp.int32)
    dn = lax.GatherDimensionNumbers(
        offset_dims=(), collapsed_slice_dims=(0,), start_index_map=(0,))
    return lax.gather(v, idx, dn, slice_sizes=(1,),
                      mode=lax.GatherScatterMode.PROMISE_IN_BOUNDS)


def _pool_body(x_hbm, batch_hbm, zmean_hbm, zmax_hbm,
               xbuf, bbuf, midbuf, valbuf, sum_acc, max_acc,
               cur_sum, cur_max, sem0, sem1):
    cid = lax.axis_index("c")
    sid = lax.axis_index("s")
    wid = sid * NC + cid
    g0 = wid * SEG_PER_W
    g0v = jnp.full((L,), 0, jnp.int32) + g0

    iota = lax.iota(jnp.int32, L)
    zero16 = jnp.zeros((L,), jnp.float32)
    ninf16 = jnp.full((L,), -jnp.inf, jnp.float32)

    def lower_bound(targets):
        def step(_, carry):
            lo, hi = carry
            mid = lax.shift_right_logical(lo + hi, 1)
            midbuf[...] = jnp.minimum(mid, N - 1)
            pltpu.sync_copy(batch_hbm.at[midbuf], valbuf)
            v = valbuf[...]
            pred = (v < targets) & (mid < N)
            lo2 = jnp.where(pred, mid + 1, lo)
            hi2 = jnp.where(pred, hi, mid)
            return lo2, hi2
        lo0 = jnp.zeros((L,), jnp.int32)
        hi0 = jnp.full((L,), N, jnp.int32)
        lo, hi = lax.fori_loop(0, 17, step, (lo0, hi0))
        return lo

    targets = g0 + iota
    starts = lower_bound(targets)
    ends = lower_bound(targets + 1)
    counts = ends - starts

    s_w = jnp.min(starts)
    e_w = jnp.max(ends)

    def init_body(i, _):
        off = i * L
        sum_acc[pl.ds(off, L)] = zero16
        max_acc[pl.ds(off, L)] = ninf16
        return 0
    lax.fori_loop(0, SEG_PER_W * DC, init_body, 0)
    for k in range(DC):
        cur_sum[pl.ds(k * L, L)] = zero16
        cur_max[pl.ds(k * L, L)] = ninf16

    cols = [iota + k * L for k in range(DC)]

    def flush_cur(cur_gv):
        mk = cur_gv >= 0
        base = (cur_gv - g0v) * D
        for k in range(DC):
            idx = base + cols[k]
            plsc.addupdate_scatter(sum_acc, [idx], cur_sum[pl.ds(k * L, L)],
                                   mask=mk)
            old = plsc.load_gather(max_acc, [idx], mask=mk)
            plsc.store_scatter(
                max_acc, [idx],
                jnp.maximum(old, cur_max[pl.ds(k * L, L)]), mask=mk)

    def group_body(gi, cur_gv):
        vb = bbuf[pl.ds(pbase_b + gi * L, L)]
        gb0 = pbase_rb + gi * 2

        def fast(cur_gv):
            for k in range(DC):
                sp = [None] * 4
                mp = [None] * 4
                for r in range(L):
                    v = xbuf[gb0 + (r >> 3), r & 7, pl.ds(k * L, L)]
                    c = r & 3
                    sp[c] = v if sp[c] is None else sp[c] + v
                    mp[c] = v if mp[c] is None else jnp.maximum(mp[c], v)
                s = (sp[0] + sp[1]) + (sp[2] + sp[3])
                m = jnp.maximum(jnp.maximum(mp[0], mp[1]),
                                jnp.maximum(mp[2], mp[3]))
                off = k * L
                cur_sum[pl.ds(off, L)] = cur_sum[pl.ds(off, L)] + s
                cur_max[pl.ds(off, L)] = jnp.maximum(cur_max[pl.ds(off, L)], m)
            return cur_gv

        def slow(cur_gv):
            flush_cur(cur_gv)
            for k in range(DC):
                cur_sum[pl.ds(k * L, L)] = zero16
                cur_max[pl.ds(k * L, L)] = ninf16

            def row_body(r, _):
                seg = _lane_bcast(vb, r) - g0v
                base = seg * D
                rbi = gb0 + lax.shift_right_logical(r, 3)
                sub = jnp.bitwise_and(r, 7)
                for k in range(DC):
                    idx = base + cols[k]
                    xc = xbuf[rbi, sub, pl.ds(k * L, L)]
                    plsc.addupdate_scatter(sum_acc, [idx], xc)
                    old = plsc.load_gather(max_acc, [idx])
                    plsc.store_scatter(max_acc, [idx], jnp.maximum(old, xc))
                return 0
            lax.fori_loop(r_lo, r_hi, row_body, 0)
            return _lane_bcast(vb, r_hi - 1)

        r_lo = jnp.maximum(j_lo - gi * L, 0)
        r_hi = jnp.minimum(j_hi - gi * L, L)
        is_full = (r_lo == 0) & (r_hi == L)
        take_fast = is_full & jnp.all(vb == cur_gv)
        return lax.cond(take_fast, fast, slow, cur_gv)

    j_lo = jnp.int32(0)
    j_hi = jnp.int32(0)
    pbase_rb = jnp.int32(0)
    pbase_b = jnp.int32(0)

    CD = C * D
    t0 = s_w // C
    t1 = (e_w + (C - 1)) // C
    t1 = jnp.where(s_w == e_w, t0, t1)

    def start_tile(t, slot, sem):
        pltpu.async_copy(x_hbm.at[pl.ds(t * TPB, TPB)],
                         xbuf.at[pl.ds(slot * TPB, TPB)], sem)
        pltpu.async_copy(batch_hbm.at[pl.ds(t * C, C)],
                         bbuf.at[pl.ds(slot * C, C)], sem)

    def wait_tile(t, slot, sem):
        pltpu.make_async_copy(x_hbm.at[pl.ds(t * TPB, TPB)],
                              xbuf.at[pl.ds(slot * TPB, TPB)], sem).wait()
        pltpu.make_async_copy(batch_hbm.at[pl.ds(t * C, C)],
                              bbuf.at[pl.ds(slot * C, C)], sem).wait()

    @pl.when(t0 < t1)
    def _():
        start_tile(t0, 0, sem0)

    def tile_body2(t, cur_gv):
        nonlocal j_lo, j_hi, pbase_rb, pbase_b
        parity = (t - t0) & 1

        @pl.when(parity == 0)
        def _():
            wait_tile(t, 0, sem0)

            @pl.when(t + 1 < t1)
            def _():
                start_tile(t + 1, 1, sem1)

        @pl.when(parity == 1)
        def _():
            wait_tile(t, 1, sem1)

            @pl.when(t + 1 < t1)
            def _():
                start_tile(t + 1, 0, sem0)

        j_lo = jnp.maximum(s_w - t * C, 0)
        j_hi = jnp.minimum(e_w - t * C, C)
        pbase_rb = parity * TPB
        pbase_b = parity * C
        gi_lo = j_lo // L
        gi_hi = (j_hi + (L - 1)) // L
        return lax.fori_loop(gi_lo, gi_hi, group_body, cur_gv)

    cur_gv = lax.fori_loop(t0, t1, tile_body2,
                           jnp.full((L,), -1, jnp.int32))
    flush_cur(cur_gv)

    countf = counts.astype(jnp.float32)

    def fin_body(g, _):
        cnt = _lane_bcast(countf, g)
        ne = cnt > 0.0
        safe = jnp.maximum(cnt, 1.0)
        for k in range(DC):
            off = g * D + k * L
            s = sum_acc[pl.ds(off, L)]
            m = max_acc[pl.ds(off, L)]
            sum_acc[pl.ds(off, L)] = jnp.where(ne, s / safe, 0.0)
            max_acc[pl.ds(off, L)] = jnp.where(ne, m, 0.0)
        return 0
    lax.fori_loop(0, SEG_PER_W, fin_body, 0)

    pltpu.sync_copy(sum_acc, zmean_hbm.at[pl.ds(g0 * D, SEG_PER_W * D)])
    pltpu.sync_copy(max_acc, zmax_hbm.at[pl.ds(g0 * D, SEG_PER_W * D)])


@jax.jit
def _pool(xf, batch):
    mesh = plsc.VectorSubcoreMesh(core_axis_name="c", subcore_axis_name="s",
                                  num_cores=NC, num_subcores=NS)
    return pl.kernel(
        _pool_body,
        compiler_params=pltpu.CompilerParams(needs_layout_passes=False),
        out_type=(
            jax.ShapeDtypeStruct((G * D,), jnp.float32),
            jax.ShapeDtypeStruct((G * D,), jnp.float32),
        ),
        mesh=mesh,
        scratch_types=[
            pltpu.VMEM((2 * TPB, 8, D), jnp.float32),
            pltpu.VMEM((2 * C,), jnp.int32),
            pltpu.VMEM((L,), jnp.int32),
            pltpu.VMEM((L,), jnp.int32),
            pltpu.VMEM((SEG_PER_W * D,), jnp.float32),
            pltpu.VMEM((SEG_PER_W * D,), jnp.float32),
            pltpu.VMEM((D,), jnp.float32),
            pltpu.VMEM((D,), jnp.float32),
            pltpu.SemaphoreType.DMA,
            pltpu.SemaphoreType.DMA,
        ],
    )(xf, batch)


def _dense_body(zm_ref, zx_ref, w1_ref, b1_ref, w2_ref, b2_ref, o_ref):
    dn = (((1,), (1,)), ((), ()))
    hm = lax.dot_general(zm_ref[...], w1_ref[...], dn,
                         preferred_element_type=jnp.float32) + b1_ref[...]
    hx = lax.dot_general(zx_ref[...], w1_ref[...], dn,
                         preferred_element_type=jnp.float32) + b1_ref[...]
    h = jnp.concatenate([hm, hx], axis=1)
    o_ref[...] = lax.dot_general(h, w2_ref[...], dn,
                                 preferred_element_type=jnp.float32) + b2_ref[...]


@jax.jit
def _dense(zm, zx, W1, b1, W2, b2):
    return pl.pallas_call(
        _dense_body,
        out_shape=jax.ShapeDtypeStruct((G, OUT), jnp.float32),
    )(zm, zx, W1, b1, W2, b2)


def kernel(x, batch, W1, b1, W2, b2):
    zm_flat, zx_flat = _pool(x.reshape(N // 8, 8, D), batch)
    zm = zm_flat.reshape(G, D)
    zx = zx_flat.reshape(G, D)
    return _dense(zm, zx, W1, b1.reshape(1, H), W2, b2.reshape(1, OUT))


# speedup vs baseline: 7.2605x; 1.0350x over previous
import functools

import jax
import jax.numpy as jnp
from jax import lax
from jax.experimental import pallas as pl
from jax.experimental.pallas import tpu as pltpu
from jax.experimental.pallas import tpu_sc as plsc

N = 100000
D = 256
G = 512
H = 128
OUT = 128

NC = 2
NS = 16
L = 16
NW = NC * NS
SEG_PER_W = G // NW
C = 160
TPB = C // 8
NT = N // C
DC = D // L


def _lane_bcast(v, i):
    idx = jnp.full((L, 1), i, dtype=jnp.int32)
    dn = lax.GatherDimensionNumbers(
        offset_dims=(), collapsed_slice_dims=(0,), start_index_map=(0,))
    return lax.gather(v, idx, dn, slice_sizes=(1,),
                      mode=lax.GatherScatterMode.PROMISE_IN_BOUNDS)


def _pool_body(x_hbm, batch_hbm, zmean_hbm, zmax_hbm,
               xbuf, bbuf, midbuf, valbuf, midbuf2, valbuf2, sum_acc,
               max_acc, cur_sum, cur_max, sem0, sem1):
    cid = lax.axis_index("c")
    sid = lax.axis_index("s")
    wid = sid * NC + cid
    g0 = wid * SEG_PER_W
    g0v = jnp.full((L,), 0, jnp.int32) + g0

    iota = lax.iota(jnp.int32, L)
    zero16 = jnp.zeros((L,), jnp.float32)
    ninf16 = jnp.full((L,), -jnp.inf, jnp.float32)

    def lower_bound_pair(tg1, tg2):
        def step(_, carry):
            lo1, hi1, lo2, hi2 = carry
            mid1 = lax.shift_right_logical(lo1 + hi1, 1)
            mid2 = lax.shift_right_logical(lo2 + hi2, 1)
            midbuf[...] = jnp.minimum(mid1, N - 1)
            midbuf2[...] = jnp.minimum(mid2, N - 1)
            d1 = pltpu.async_copy(batch_hbm.at[midbuf], valbuf, sem0)
            d2 = pltpu.async_copy(batch_hbm.at[midbuf2], valbuf2, sem1)
            d1.wait()
            d2.wait()
            v1 = valbuf[...]
            v2 = valbuf2[...]
            p1 = (v1 < tg1) & (mid1 < N)
            p2 = (v2 < tg2) & (mid2 < N)
            return (jnp.where(p1, mid1 + 1, lo1), jnp.where(p1, hi1, mid1),
                    jnp.where(p2, mid2 + 1, lo2), jnp.where(p2, hi2, mid2))
        z = jnp.zeros((L,), jnp.int32)
        f = jnp.full((L,), N, jnp.int32)
        lo1, _, lo2, _ = lax.fori_loop(0, 17, step, (z, f, z, f))
        return lo1, lo2

    targets = g0 + iota
    starts, ends = lower_bound_pair(targets, targets + 1)
    counts = ends - starts

    s_w = jnp.min(starts)
    e_w = jnp.max(ends)

    def init_body(i, _):
        off = i * L
        sum_acc[pl.ds(off, L)] = zero16
        max_acc[pl.ds(off, L)] = ninf16
        return 0
    lax.fori_loop(0, SEG_PER_W * DC, init_body, 0)
    for k in range(DC):
        cur_sum[pl.ds(k * L, L)] = zero16
        cur_max[pl.ds(k * L, L)] = ninf16

    cols = [iota + k * L for k in range(DC)]

    def flush_cur(cur_gv):
        mk = cur_gv >= 0
        base = (cur_gv - g0v) * D
        for k in range(DC):
            idx = base + cols[k]
            plsc.addupdate_scatter(sum_acc, [idx], cur_sum[pl.ds(k * L, L)],
                                   mask=mk)
            old = plsc.load_gather(max_acc, [idx], mask=mk)
            plsc.store_scatter(
                max_acc, [idx],
                jnp.maximum(old, cur_max[pl.ds(k * L, L)]), mask=mk)

    def group_body(gi, cur_gv):
        vb = bbuf[pl.ds(pbase_b + gi * L, L)]
        gb0 = pbase_rb + gi * 2

        def fast(cur_gv):
            for k in range(DC):
                sp = [None] * 4
                mp = [None] * 4
                for r in range(L):
                    v = xbuf[gb0 + (r >> 3), r & 7, pl.ds(k * L, L)]
                    c = r & 3
                    sp[c] = v if sp[c] is None else sp[c] + v
                    mp[c] = v if mp[c] is None else jnp.maximum(mp[c], v)
                s = (sp[0] + sp[1]) + (sp[2] + sp[3])
                m = jnp.maximum(jnp.maximum(mp[0], mp[1]),
                                jnp.maximum(mp[2], mp[3]))
                off = k * L
                cur_sum[pl.ds(off, L)] = cur_sum[pl.ds(off, L)] + s
                cur_max[pl.ds(off, L)] = jnp.maximum(cur_max[pl.ds(off, L)], m)
            return cur_gv

        def slow(cur_gv):
            flush_cur(cur_gv)
            for k in range(DC):
                cur_sum[pl.ds(k * L, L)] = zero16
                cur_max[pl.ds(k * L, L)] = ninf16

            def row_body(r, _):
                seg = _lane_bcast(vb, r) - g0v
                base = seg * D
                rbi = gb0 + lax.shift_right_logical(r, 3)
                sub = jnp.bitwise_and(r, 7)
                for k in range(DC):
                    idx = base + cols[k]
                    xc = xbuf[rbi, sub, pl.ds(k * L, L)]
                    plsc.addupdate_scatter(sum_acc, [idx], xc)
                    old = plsc.load_gather(max_acc, [idx])
                    plsc.store_scatter(max_acc, [idx], jnp.maximum(old, xc))
                return 0
            lax.fori_loop(r_lo, r_hi, row_body, 0)
            return _lane_bcast(vb, r_hi - 1)

        r_lo = jnp.maximum(j_lo - gi * L, 0)
        r_hi = jnp.minimum(j_hi - gi * L, L)
        is_full = (r_lo == 0) & (r_hi == L)
        take_fast = is_full & jnp.all(vb == cur_gv)
        return lax.cond(take_fast, fast, slow, cur_gv)

    j_lo = jnp.int32(0)
    j_hi = jnp.int32(0)
    pbase_rb = jnp.int32(0)
    pbase_b = jnp.int32(0)

    CD = C * D
    t0 = s_w // C
    t1 = (e_w + (C - 1)) // C
    t1 = jnp.where(s_w == e_w, t0, t1)

    def start_tile(t, slot, sem):
        pltpu.async_copy(x_hbm.at[pl.ds(t * TPB, TPB)],
                         xbuf.at[pl.ds(slot * TPB, TPB)], sem)
        pltpu.async_copy(batch_hbm.at[pl.ds(t * C, C)],
                         bbuf.at[pl.ds(slot * C, C)], sem)

    def wait_tile(t, slot, sem):
        pltpu.make_async_copy(x_hbm.at[pl.ds(t * TPB, TPB)],
                              xbuf.at[pl.ds(slot * TPB, TPB)], sem).wait()
        pltpu.make_async_copy(batch_hbm.at[pl.ds(t * C, C)],
                              bbuf.at[pl.ds(slot * C, C)], sem).wait()

    @pl.when(t0 < t1)
    def _():
        start_tile(t0, 0, sem0)

    def tile_body2(t, cur_gv):
        nonlocal j_lo, j_hi, pbase_rb, pbase_b
        parity = (t - t0) & 1

        @pl.when(parity == 0)
        def _():
            wait_tile(t, 0, sem0)

            @pl.when(t + 1 < t1)
            def _():
                start_tile(t + 1, 1, sem1)

        @pl.when(parity == 1)
        def _():
            wait_tile(t, 1, sem1)

            @pl.when(t + 1 < t1)
            def _():
                start_tile(t + 1, 0, sem0)

        j_lo = jnp.maximum(s_w - t * C, 0)
        j_hi = jnp.minimum(e_w - t * C, C)
        pbase_rb = parity * TPB
        pbase_b = parity * C
        gi_lo = j_lo // L
        gi_hi = (j_hi + (L - 1)) // L
        return lax.fori_loop(gi_lo, gi_hi, group_body, cur_gv)

    cur_gv = lax.fori_loop(t0, t1, tile_body2,
                           jnp.full((L,), -1, jnp.int32))
    flush_cur(cur_gv)

    countf = counts.astype(jnp.float32)

    def fin_body(g, _):
        cnt = _lane_bcast(countf, g)
        ne = cnt > 0.0
        safe = jnp.maximum(cnt, 1.0)
        for k in range(DC):
            off = g * D + k * L
            s = sum_acc[pl.ds(off, L)]
            m = max_acc[pl.ds(off, L)]
            sum_acc[pl.ds(off, L)] = jnp.where(ne, s / safe, 0.0)
            max_acc[pl.ds(off, L)] = jnp.where(ne, m, 0.0)
        return 0
    lax.fori_loop(0, SEG_PER_W, fin_body, 0)

    pltpu.sync_copy(sum_acc, zmean_hbm.at[pl.ds(g0 * D, SEG_PER_W * D)])
    pltpu.sync_copy(max_acc, zmax_hbm.at[pl.ds(g0 * D, SEG_PER_W * D)])


@jax.jit
def _pool(xf, batch):
    mesh = plsc.VectorSubcoreMesh(core_axis_name="c", subcore_axis_name="s",
                                  num_cores=NC, num_subcores=NS)
    return pl.kernel(
        _pool_body,
        compiler_params=pltpu.CompilerParams(needs_layout_passes=False),
        out_type=(
            jax.ShapeDtypeStruct((G * D,), jnp.float32),
            jax.ShapeDtypeStruct((G * D,), jnp.float32),
        ),
        mesh=mesh,
        scratch_types=[
            pltpu.VMEM((2 * TPB, 8, D), jnp.float32),
            pltpu.VMEM((2 * C,), jnp.int32),
            pltpu.VMEM((L,), jnp.int32),
            pltpu.VMEM((L,), jnp.int32),
            pltpu.VMEM((L,), jnp.int32),
            pltpu.VMEM((L,), jnp.int32),
            pltpu.VMEM((SEG_PER_W * D,), jnp.float32),
            pltpu.VMEM((SEG_PER_W * D,), jnp.float32),
            pltpu.VMEM((D,), jnp.float32),
            pltpu.VMEM((D,), jnp.float32),
            pltpu.SemaphoreType.DMA,
            pltpu.SemaphoreType.DMA,
        ],
    )(xf, batch)


def _dense_body(zm_ref, zx_ref, w1_ref, b1_ref, w2_ref, b2_ref, o_ref):
    dn = (((1,), (1,)), ((), ()))
    hm = lax.dot_general(zm_ref[...], w1_ref[...], dn,
                         preferred_element_type=jnp.float32) + b1_ref[...]
    hx = lax.dot_general(zx_ref[...], w1_ref[...], dn,
                         preferred_element_type=jnp.float32) + b1_ref[...]
    h = jnp.concatenate([hm, hx], axis=1)
    o_ref[...] = lax.dot_general(h, w2_ref[...], dn,
                                 preferred_element_type=jnp.float32) + b2_ref[...]


@jax.jit
def _dense(zm, zx, W1, b1, W2, b2):
    return pl.pallas_call(
        _dense_body,
        out_shape=jax.ShapeDtypeStruct((G, OUT), jnp.float32),
    )(zm, zx, W1, b1, W2, b2)


def kernel(x, batch, W1, b1, W2, b2):
    zm_flat, zx_flat = _pool(x.reshape(N // 8, 8, D), batch)
    zm = zm_flat.reshape(G, D)
    zx = zx_flat.reshape(G, D)
    return _dense(zm, zx, W1, b1.reshape(1, H), W2, b2.reshape(1, OUT))
